# Initial kernel scaffold; baseline (speedup 1.0000x reference)
import jax
import jax.numpy as jnp
from jax.experimental import pallas as pl


def kernel(node_h, edge_attr, batch, edge_index, W1, b1, Wl, bl, eps_gine, gn_weight, gn_bias, gn_mean_scale):
    raise NotImplementedError("write your pallas kernel here")


# speedup vs baseline: 2.5804x; 2.5804x over previous
import functools

import jax
import jax.numpy as jnp
from jax import lax
from jax.experimental import pallas as pl
from jax.experimental.pallas import tpu as pltpu
from jax.experimental.pallas import tpu_sc as plsc

N = 10000
E = 320000
D = 128
G = 16

NC = 2
NS = 16
NW = NC * NS
EPT = E // NW
C = 80
NCHUNK = EPT // C
RZ = 200
NRC = N // RZ
KMAX = (NRC + NS - 1) // NS
NB = 5
R = N // NB


def _mm_bias_body(x_ref, w_ref, b_ref, o_ref):
    o_ref[...] = lax.dot_general(
        x_ref[...], w_ref[...], (((1,), (1,)), ((), ())),
        preferred_element_type=jnp.float32) + b_ref[...]


def _node_mm(node_h, W1, b1):
    return pl.pallas_call(
        _mm_bias_body,
        out_shape=jax.ShapeDtypeStruct((N, D), jnp.float32),
    )(node_h, W1, b1.reshape(1, D))


def _edge_mm(edge_attr, Wl, bl):
    BE = 4000
    return pl.pallas_call(
        _mm_bias_body,
        grid=(E // BE,),
        in_specs=[
            pl.BlockSpec((BE, D), lambda i: (i, 0)),
            pl.BlockSpec((D, D), lambda i: (0, 0)),
            pl.BlockSpec((1, D), lambda i: (0, 0)),
        ],
        out_specs=pl.BlockSpec((BE, D), lambda i: (i, 0)),
        out_shape=jax.ShapeDtypeStruct((E, D), jnp.float32),
    )(edge_attr, Wl, bl.reshape(1, D))


def _sc_agg_body(h_hbm, e_hbm, src_hbm, dst_hbm, out_hbm,
                 src_v, dst_v, e_v, g_v, zbuf, agg_sh, sem):
    cid = lax.axis_index("c")
    sid = lax.axis_index("s")
    wid = cid * NS + sid

    def _zrow(r, carry):
        for j in range(D // 16):
            zbuf[r, pl.ds(16 * j, 16)] = jnp.zeros((16,), jnp.float32)
        return carry
    lax.fori_loop(0, RZ, _zrow, 0)
    for k in range(KMAX):
        rc = sid + k * NS

        @pl.when(rc < NRC)
        def _():
            pltpu.sync_copy(zbuf, agg_sh.at[pl.ds(rc * RZ, RZ), :])
    plsc.subcore_barrier()

    ebase = wid * EPT

    def _chunk(it, carry):
        base = ebase + it * C
        pltpu.sync_copy(src_hbm.at[pl.ds(base, C)], src_v)
        pltpu.sync_copy(dst_hbm.at[pl.ds(base, C)], dst_v)
        pltpu.sync_copy(e_hbm.at[pl.ds(base, C), :], e_v)
        pltpu.async_copy(h_hbm.at[src_v], g_v, sem).wait()

        def _row(r, c2):
            for j in range(D // 16):
                sl = pl.ds(16 * j, 16)
                x = g_v[r, sl] + e_v[r, sl]
                g_v[r, sl] = x / (1.0 + jnp.exp(-x))
            return c2
        lax.fori_loop(0, C, _row, 0)
        pltpu.sync_copy(g_v, agg_sh.at[dst_v], add=True)
        return carry
    lax.fori_loop(0, NCHUNK, _chunk, 0)

    plsc.subcore_barrier()
    for k in range(KMAX):
        rc = sid + k * NS

        @pl.when(rc < NRC)
        def _():
            pltpu.sync_copy(agg_sh.at[pl.ds(rc * RZ, RZ), :], zbuf)
            pltpu.sync_copy(zbuf, out_hbm.at[cid, pl.ds(rc * RZ, RZ), :])


_sc_agg = functools.partial(
    pl.kernel,
    out_type=jax.ShapeDtypeStruct((NC, N, D), jnp.float32),
    mesh=plsc.VectorSubcoreMesh(core_axis_name="c", subcore_axis_name="s"),
    scratch_types=[
        pltpu.VMEM((C,), jnp.int32),
        pltpu.VMEM((C,), jnp.int32),
        pltpu.VMEM((C, D), jnp.float32),
        pltpu.VMEM((C, D), jnp.float32),
        pltpu.VMEM((RZ, D), jnp.float32),
        pltpu.VMEM_SHARED((N, D), jnp.float32),
        pltpu.SemaphoreType.DMA,
    ],
)(_sc_agg_body)


def _c1_body(p_ref, h_ref, h0_ref, b_ref, eps_ref,
             y_ref, s1_ref, sq_ref, cnt_ref):
    i = pl.program_id(0)
    agg = p_ref[0] + p_ref[1]
    x = agg + (1.0 + eps_ref[0, 0]) * h_ref[...]
    x = x * (1.0 / (1.0 + jnp.exp(-x)))
    y = x + h0_ref[...]
    y_ref[...] = y
    b = b_ref[0, 0, :]
    onehot = (b[:, None] == lax.broadcasted_iota(jnp.int32, (1, G), 1)
              ).astype(jnp.float32)
    s1 = lax.dot_general(onehot, y, (((0,), (0,)), ((), ())),
                         preferred_element_type=jnp.float32)
    sq = lax.dot_general(onehot, y * y, (((0,), (0,)), ((), ())),
                         preferred_element_type=jnp.float32)
    cnt = lax.dot_general(onehot, jnp.ones((R, D), jnp.float32),
                          (((0,), (0,)), ((), ())),
                          preferred_element_type=jnp.float32)

    @pl.when(i == 0)
    def _():
        s1_ref[...] = jnp.zeros_like(s1_ref)
        sq_ref[...] = jnp.zeros_like(sq_ref)
        cnt_ref[...] = jnp.zeros_like(cnt_ref)

    s1_ref[...] += s1
    sq_ref[...] += sq
    cnt_ref[...] += cnt


def _c2_body(y_ref, b_ref, s1_ref, sq_ref, cnt_ref, sc_ref, w_ref, bias_ref,
             o_ref):
    n = jnp.maximum(cnt_ref[...], 1.0)
    mean = s1_ref[...] / n
    ms = mean * sc_ref[...]
    var = sq_ref[...] / n - 2.0 * ms * mean + ms * ms
    winv = w_ref[...] * lax.rsqrt(var + 1e-5)
    b = b_ref[0, 0, :]
    onehot = (b[:, None] == lax.broadcasted_iota(jnp.int32, (1, G), 1)
              ).astype(jnp.float32)
    ms_r = lax.dot_general(onehot, ms, (((1,), (0,)), ((), ())),
                           preferred_element_type=jnp.float32)
    winv_r = lax.dot_general(onehot, winv, (((1,), (0,)), ((), ())),
                             preferred_element_type=jnp.float32)
    o_ref[...] = (y_ref[...] - ms_r) * winv_r + bias_ref[...]


def _node_phase(parts, h, node_h, batch, eps_gine,
                gn_weight, gn_bias, gn_mean_scale):
    batch3 = batch.reshape(NB, 1, R)
    eps2 = eps_gine.reshape(1, 1)
    y, s1, sq, cnt = pl.pallas_call(
        _c1_body,
        grid=(NB,),
        in_specs=[
            pl.BlockSpec((NC, R, D), lambda i: (0, i, 0)),
            pl.BlockSpec((R, D), lambda i: (i, 0)),
            pl.BlockSpec((R, D), lambda i: (i, 0)),
            pl.BlockSpec((1, 1, R), lambda i: (i, 0, 0)),
            pl.BlockSpec((1, 1), lambda i: (0, 0)),
        ],
        out_specs=[
            pl.BlockSpec((R, D), lambda i: (i, 0)),
            pl.BlockSpec((G, D), lambda i: (0, 0)),
            pl.BlockSpec((G, D), lambda i: (0, 0)),
            pl.BlockSpec((G, D), lambda i: (0, 0)),
        ],
        out_shape=[
            jax.ShapeDtypeStruct((N, D), jnp.float32),
            jax.ShapeDtypeStruct((G, D), jnp.float32),
            jax.ShapeDtypeStruct((G, D), jnp.float32),
            jax.ShapeDtypeStruct((G, D), jnp.float32),
        ],
    )(parts, h, node_h, batch3, eps2)

    return pl.pallas_call(
        _c2_body,
        grid=(NB,),
        in_specs=[
            pl.BlockSpec((R, D), lambda i: (i, 0)),
            pl.BlockSpec((1, 1, R), lambda i: (i, 0, 0)),
            pl.BlockSpec((G, D), lambda i: (0, 0)),
            pl.BlockSpec((G, D), lambda i: (0, 0)),
            pl.BlockSpec((G, D), lambda i: (0, 0)),
            pl.BlockSpec((1, D), lambda i: (0, 0)),
            pl.BlockSpec((1, D), lambda i: (0, 0)),
            pl.BlockSpec((1, D), lambda i: (0, 0)),
        ],
        out_specs=pl.BlockSpec((R, D), lambda i: (i, 0)),
        out_shape=jax.ShapeDtypeStruct((N, D), jnp.float32),
    )(y, batch3, s1, sq, cnt, gn_mean_scale.reshape(1, D),
      gn_weight.reshape(1, D), gn_bias.reshape(1, D))


def kernel(node_h, edge_attr, batch, edge_index, W1, b1, Wl, bl, eps_gine,
           gn_weight, gn_bias, gn_mean_scale):
    h = _node_mm(node_h, W1, b1)
    e = _edge_mm(edge_attr, Wl, bl)
    parts = _sc_agg(h, e, edge_index[0], edge_index[1])
    return _node_phase(parts, h, node_h, batch, eps_gine,
                       gn_weight, gn_bias, gn_mean_scale)


# speedup vs baseline: 4.8439x; 1.8772x over previous
import functools

import jax
import jax.numpy as jnp
from jax import lax
from jax.experimental import pallas as pl
from jax.experimental.pallas import tpu as pltpu
from jax.experimental.pallas import tpu_sc as plsc

N = 10000
E = 320000
D = 128
G = 16

NC = 2
NS = 16
NW = NC * NS
EPT = E // NW
C = 40
NCHUNK = EPT // C
RZ = 40
NRC = N // RZ
KMAX = (NRC + NS - 1) // NS
NB = 5
R = N // NB


def _mm_bias_body(x_ref, w_ref, b_ref, o_ref):
    o_ref[...] = lax.dot_general(
        x_ref[...], w_ref[...], (((1,), (1,)), ((), ())),
        preferred_element_type=jnp.float32) + b_ref[...]


def _node_mm(node_h, W1, b1):
    return pl.pallas_call(
        _mm_bias_body,
        out_shape=jax.ShapeDtypeStruct((N, D), jnp.float32),
    )(node_h, W1, b1.reshape(1, D))


def _edge_mm(edge_attr, Wl, bl):
    BE = 4000
    return pl.pallas_call(
        _mm_bias_body,
        grid=(E // BE,),
        in_specs=[
            pl.BlockSpec((BE, D), lambda i: (i, 0)),
            pl.BlockSpec((D, D), lambda i: (0, 0)),
            pl.BlockSpec((1, D), lambda i: (0, 0)),
        ],
        out_specs=pl.BlockSpec((BE, D), lambda i: (i, 0)),
        out_shape=jax.ShapeDtypeStruct((E, D), jnp.float32),
    )(edge_attr, Wl, bl.reshape(1, D))


def _sc_agg_body(h_hbm, e_hbm, src_hbm, dst_hbm, out_hbm,
                 src0, src1, dst0, dst1, dc0, dc1, dc2, dc3,
                 e0, e1, e2, e3, g0, g1, g2, g3, agg_sh,
                 is0, is1, id0, id1,
                 ee0, ee1, ee2, ee3, ge0, ge1, ge2, ge3,
                 se0, se1, se2, se3):
    srcv = (src0, src1)
    dstv = (dst0, dst1)
    dscat = (dc0, dc1, dc2, dc3)
    ev = (e0, e1, e2, e3)
    gv = (g0, g1, g2, g3)
    isem = (is0, is1)
    idsem = (id0, id1)
    eesem = (ee0, ee1, ee2, ee3)
    gesem = (ge0, ge1, ge2, ge3)
    sesem = (se0, se1, se2, se3)

    cid = lax.axis_index("c")
    sid = lax.axis_index("s")
    wid = cid * NS + sid
    ebase = wid * EPT

    def _zrow(r, carry):
        for t in range(D // 16):
            g0[r, pl.ds(16 * t, 16)] = jnp.zeros((16,), jnp.float32)
        return carry
    lax.fori_loop(0, C, _zrow, 0)
    for k in range(KMAX):
        rc = sid + k * NS

        @pl.when(rc < NRC)
        def _():
            pltpu.sync_copy(g0, agg_sh.at[pl.ds(rc * RZ, RZ), :])
    plsc.subcore_barrier()

    def issue_idx(j, s):
        off = ebase + j * C
        pltpu.async_copy(src_hbm.at[pl.ds(off, C)], srcv[s], isem[s])
        pltpu.async_copy(dst_hbm.at[pl.ds(off, C)], dstv[s], idsem[s])

    def wait_idx(s):
        pltpu.make_async_copy(src_hbm.at[pl.ds(0, C)], srcv[s], isem[s]).wait()
        pltpu.make_async_copy(dst_hbm.at[pl.ds(0, C)], dstv[s], idsem[s]).wait()

    def issue_e(j, u):
        off = ebase + j * C
        pltpu.async_copy(e_hbm.at[pl.ds(off, C), :], ev[u], eesem[u])

    def wait_e(u):
        pltpu.make_async_copy(e_hbm.at[pl.ds(0, C), :], ev[u], eesem[u]).wait()

    def issue_gather(u, s):
        pltpu.async_copy(h_hbm.at[srcv[s]], gv[u], gesem[u])

    def wait_gather(u, s):
        pltpu.make_async_copy(h_hbm.at[srcv[s]], gv[u], gesem[u]).wait()

    def issue_scatter(u):
        pltpu.async_copy(gv[u], agg_sh.at[dscat[u]], sesem[u], add=True)

    def wait_scatter(u):
        pltpu.make_async_copy(gv[u], agg_sh.at[dscat[u]], sesem[u]).wait()

    def compute(u):
        def _row(r, c2):
            for t in range(D // 16):
                sl = pl.ds(16 * t, 16)
                x = gv[u][r, sl] + ev[u][r, sl]
                gv[u][r, sl] = x / (1.0 + jnp.exp(-x))
            return c2
        lax.fori_loop(0, C, _row, 0)

    issue_idx(0, 0)
    issue_idx(1, 1)
    wait_idx(0)
    issue_gather(0, 0)
    issue_e(0, 0)

    _offs = list(range(0, C - 15, 16)) + ([C - 16] if C % 16 else [])

    def _stage_dst(u, s):
        for o in _offs:
            sl = pl.ds(o, 16)
            dscat[u][sl] = dstv[s][sl]

    def _quad(it, carry):
        for u in range(4):
            j = it * 4 + u
            q = (u + 1) % 4
            wait_gather(u, u % 2)
            wait_e(u)
            _stage_dst(u, u % 2)
            issue_idx(j + 2, u % 2)
            wait_idx((u + 1) % 2)
            issue_e(j + 1, q)

            @pl.when(j >= 3)
            def _():
                wait_scatter(q)

            issue_gather(q, (u + 1) % 2)
            compute(u)
            issue_scatter(u)
        return carry
    lax.fori_loop(0, (NCHUNK - 2) // 4, _quad, 0)

    wait_gather(0, 0)
    wait_e(0)
    _stage_dst(0, 0)
    wait_idx(1)
    issue_e(NCHUNK - 1, 1)
    wait_scatter(1)
    issue_gather(1, 1)
    compute(0)
    issue_scatter(0)
    wait_gather(1, 1)
    wait_e(1)
    _stage_dst(1, 1)
    compute(1)
    issue_scatter(1)
    for u in (2, 3, 0, 1):
        wait_scatter(u)

    plsc.subcore_barrier()
    for k in range(KMAX):
        rc = sid + k * NS

        @pl.when(rc < NRC)
        def _():
            pltpu.sync_copy(agg_sh.at[pl.ds(rc * RZ, RZ), :], g0)
            pltpu.sync_copy(g0, out_hbm.at[cid, pl.ds(rc * RZ, RZ), :])


_sc_agg = functools.partial(
    pl.kernel,
    out_type=jax.ShapeDtypeStruct((NC, N, D), jnp.float32),
    mesh=plsc.VectorSubcoreMesh(core_axis_name="c", subcore_axis_name="s"),
    scratch_types=(
        [pltpu.VMEM((C,), jnp.int32)] * 8
        + [pltpu.VMEM((C, D), jnp.float32)] * 8
        + [pltpu.VMEM_SHARED((N, D), jnp.float32)]
        + [pltpu.SemaphoreType.DMA] * 16
    ),
)(_sc_agg_body)


def _c1_body(p_ref, h_ref, h0_ref, b_ref, eps_ref,
             y_ref, s1_ref, sq_ref, cnt_ref):
    i = pl.program_id(0)
    agg = p_ref[0] + p_ref[1]
    x = agg + (1.0 + eps_ref[0, 0]) * h_ref[...]
    x = x * (1.0 / (1.0 + jnp.exp(-x)))
    y = x + h0_ref[...]
    y_ref[...] = y
    b = b_ref[0, 0, :]
    onehot = (b[:, None] == lax.broadcasted_iota(jnp.int32, (1, G), 1)
              ).astype(jnp.float32)
    s1 = lax.dot_general(onehot, y, (((0,), (0,)), ((), ())),
                         preferred_element_type=jnp.float32)
    sq = lax.dot_general(onehot, y * y, (((0,), (0,)), ((), ())),
                         preferred_element_type=jnp.float32)
    cnt = lax.dot_general(onehot, jnp.ones((R, D), jnp.float32),
                          (((0,), (0,)), ((), ())),
                          preferred_element_type=jnp.float32)

    @pl.when(i == 0)
    def _():
        s1_ref[...] = jnp.zeros_like(s1_ref)
        sq_ref[...] = jnp.zeros_like(sq_ref)
        cnt_ref[...] = jnp.zeros_like(cnt_ref)

    s1_ref[...] += s1
    sq_ref[...] += sq
    cnt_ref[...] += cnt


def _c2_body(y_ref, b_ref, s1_ref, sq_ref, cnt_ref, sc_ref, w_ref, bias_ref,
             o_ref):
    n = jnp.maximum(cnt_ref[...], 1.0)
    mean = s1_ref[...] / n
    ms = mean * sc_ref[...]
    var = sq_ref[...] / n - 2.0 * ms * mean + ms * ms
    winv = w_ref[...] * lax.rsqrt(var + 1e-5)
    b = b_ref[0, 0, :]
    onehot = (b[:, None] == lax.broadcasted_iota(jnp.int32, (1, G), 1)
              ).astype(jnp.float32)
    ms_r = lax.dot_general(onehot, ms, (((1,), (0,)), ((), ())),
                           preferred_element_type=jnp.float32)
    winv_r = lax.dot_general(onehot, winv, (((1,), (0,)), ((), ())),
                             preferred_element_type=jnp.float32)
    o_ref[...] = (y_ref[...] - ms_r) * winv_r + bias_ref[...]


def _node_phase(parts, h, node_h, batch, eps_gine,
                gn_weight, gn_bias, gn_mean_scale):
    batch3 = batch.reshape(NB, 1, R)
    eps2 = eps_gine.reshape(1, 1)
    y, s1, sq, cnt = pl.pallas_call(
        _c1_body,
        grid=(NB,),
        in_specs=[
            pl.BlockSpec((NC, R, D), lambda i: (0, i, 0)),
            pl.BlockSpec((R, D), lambda i: (i, 0)),
            pl.BlockSpec((R, D), lambda i: (i, 0)),
            pl.BlockSpec((1, 1, R), lambda i: (i, 0, 0)),
            pl.BlockSpec((1, 1), lambda i: (0, 0)),
        ],
        out_specs=[
            pl.BlockSpec((R, D), lambda i: (i, 0)),
            pl.BlockSpec((G, D), lambda i: (0, 0)),
            pl.BlockSpec((G, D), lambda i: (0, 0)),
            pl.BlockSpec((G, D), lambda i: (0, 0)),
        ],
        out_shape=[
            jax.ShapeDtypeStruct((N, D), jnp.float32),
            jax.ShapeDtypeStruct((G, D), jnp.float32),
            jax.ShapeDtypeStruct((G, D), jnp.float32),
            jax.ShapeDtypeStruct((G, D), jnp.float32),
        ],
    )(parts, h, node_h, batch3, eps2)

    return pl.pallas_call(
        _c2_body,
        grid=(NB,),
        in_specs=[
            pl.BlockSpec((R, D), lambda i: (i, 0)),
            pl.BlockSpec((1, 1, R), lambda i: (i, 0, 0)),
            pl.BlockSpec((G, D), lambda i: (0, 0)),
            pl.BlockSpec((G, D), lambda i: (0, 0)),
            pl.BlockSpec((G, D), lambda i: (0, 0)),
            pl.BlockSpec((1, D), lambda i: (0, 0)),
            pl.BlockSpec((1, D), lambda i: (0, 0)),
            pl.BlockSpec((1, D), lambda i: (0, 0)),
        ],
        out_specs=pl.BlockSpec((R, D), lambda i: (i, 0)),
        out_shape=jax.ShapeDtypeStruct((N, D), jnp.float32),
    )(y, batch3, s1, sq, cnt, gn_mean_scale.reshape(1, D),
      gn_weight.reshape(1, D), gn_bias.reshape(1, D))


def kernel(node_h, edge_attr, batch, edge_index, W1, b1, Wl, bl, eps_gine,
           gn_weight, gn_bias, gn_mean_scale):
    h = _node_mm(node_h, W1, b1)
    e = _edge_mm(edge_attr, Wl, bl)
    parts = _sc_agg(h, e, edge_index[0], edge_index[1])
    return _node_phase(parts, h, node_h, batch, eps_gine,
                       gn_weight, gn_bias, gn_mean_scale)


# speedup vs baseline: 4.8875x; 1.0090x over previous
import functools

import jax
import jax.numpy as jnp
from jax import lax
from jax.experimental import pallas as pl
from jax.experimental.pallas import tpu as pltpu
from jax.experimental.pallas import tpu_sc as plsc

N = 10000
E = 320000
D = 128
G = 16

NC = 2
NS = 16
NW = NC * NS
EPT = E // NW
C = 40
NCHUNK = EPT // C
RZ = 40
NRC = N // RZ
KMAX = (NRC + NS - 1) // NS
NB = 5
R = N // NB


def _mm_bias_body(x_ref, w_ref, b_ref, o_ref):
    o_ref[...] = lax.dot_general(
        x_ref[...], w_ref[...], (((1,), (1,)), ((), ())),
        preferred_element_type=jnp.float32) + b_ref[...]


def _node_mm(node_h, W1, b1):
    return pl.pallas_call(
        _mm_bias_body,
        out_shape=jax.ShapeDtypeStruct((N, D), jnp.float32),
    )(node_h, W1, b1.reshape(1, D))


def _edge_mm(edge_attr, Wl, bl):
    BE = 4000
    return pl.pallas_call(
        _mm_bias_body,
        grid=(E // BE,),
        in_specs=[
            pl.BlockSpec((BE, D), lambda i: (i, 0)),
            pl.BlockSpec((D, D), lambda i: (0, 0)),
            pl.BlockSpec((1, D), lambda i: (0, 0)),
        ],
        out_specs=pl.BlockSpec((BE, D), lambda i: (i, 0)),
        out_shape=jax.ShapeDtypeStruct((E, D), jnp.float32),
    )(edge_attr, Wl, bl.reshape(1, D))


def _sc_agg_body(h_hbm, e_hbm, src_hbm, dst_hbm, out_hbm,
                 src0, src1, dst0, dst1, dc0, dc1, dc2, dc3,
                 e0, e1, e2, e3, g0, g1, g2, g3, agg_sh,
                 is0, is1, id0, id1,
                 ee0, ee1, ee2, ee3, ge0, ge1, ge2, ge3,
                 se0, se1, se2, se3):
    srcv = (src0, src1)
    dstv = (dst0, dst1)
    dscat = (dc0, dc1, dc2, dc3)
    ev = (e0, e1, e2, e3)
    gv = (g0, g1, g2, g3)
    isem = (is0, is1)
    idsem = (id0, id1)
    eesem = (ee0, ee1, ee2, ee3)
    gesem = (ge0, ge1, ge2, ge3)
    sesem = (se0, se1, se2, se3)

    cid = lax.axis_index("c")
    sid = lax.axis_index("s")
    wid = cid * NS + sid
    ebase = wid * EPT

    def _zrow(r, carry):
        for t in range(D // 16):
            g0[r, pl.ds(16 * t, 16)] = jnp.zeros((16,), jnp.float32)
        return carry
    lax.fori_loop(0, C, _zrow, 0)
    for k in range(KMAX):
        rc = sid + k * NS

        @pl.when(rc < NRC)
        def _():
            pltpu.async_copy(g0, agg_sh.at[pl.ds(rc * RZ, RZ), :], is0)
    for k in range(KMAX):
        rc = sid + k * NS

        @pl.when(rc < NRC)
        def _():
            pltpu.make_async_copy(g0, agg_sh.at[pl.ds(0, RZ), :], is0).wait()
    plsc.subcore_barrier()

    def issue_idx(j, s):
        off = ebase + j * C
        pltpu.async_copy(src_hbm.at[pl.ds(off, C)], srcv[s], isem[s])
        pltpu.async_copy(dst_hbm.at[pl.ds(off, C)], dstv[s], idsem[s])

    def wait_idx(s):
        pltpu.make_async_copy(src_hbm.at[pl.ds(0, C)], srcv[s], isem[s]).wait()
        pltpu.make_async_copy(dst_hbm.at[pl.ds(0, C)], dstv[s], idsem[s]).wait()

    def issue_e(j, u):
        off = ebase + j * C
        pltpu.async_copy(e_hbm.at[pl.ds(off, C), :], ev[u], eesem[u])

    def wait_e(u):
        pltpu.make_async_copy(e_hbm.at[pl.ds(0, C), :], ev[u], eesem[u]).wait()

    def issue_gather(u, s):
        pltpu.async_copy(h_hbm.at[srcv[s]], gv[u], gesem[u])

    def wait_gather(u, s):
        pltpu.make_async_copy(h_hbm.at[srcv[s]], gv[u], gesem[u]).wait()

    def issue_scatter(u):
        pltpu.async_copy(gv[u], agg_sh.at[dscat[u]], sesem[u], add=True)

    def wait_scatter(u):
        pltpu.make_async_copy(gv[u], agg_sh.at[dscat[u]], sesem[u]).wait()

    def compute(u):
        def _row(r, c2):
            for t in range(D // 16):
                sl = pl.ds(16 * t, 16)
                x = gv[u][r, sl] + ev[u][r, sl]
                gv[u][r, sl] = x / (1.0 + jnp.exp(-x))
            return c2
        lax.fori_loop(0, C, _row, 0)

    issue_idx(0, 0)
    issue_idx(1, 1)
    wait_idx(0)
    issue_gather(0, 0)
    issue_e(0, 0)

    _offs = list(range(0, C - 15, 16)) + ([C - 16] if C % 16 else [])

    def _stage_dst(u, s):
        for o in _offs:
            sl = pl.ds(o, 16)
            dscat[u][sl] = dstv[s][sl]

    def _quad(it, carry):
        for u in range(4):
            j = it * 4 + u
            q = (u + 1) % 4
            wait_gather(u, u % 2)
            wait_e(u)
            _stage_dst(u, u % 2)
            issue_idx(j + 2, u % 2)
            wait_idx((u + 1) % 2)
            issue_e(j + 1, q)

            @pl.when(j >= 3)
            def _():
                wait_scatter(q)

            issue_gather(q, (u + 1) % 2)
            compute(u)
            issue_scatter(u)
        return carry
    lax.fori_loop(0, (NCHUNK - 2) // 4, _quad, 0)

    wait_gather(0, 0)
    wait_e(0)
    _stage_dst(0, 0)
    wait_idx(1)
    issue_e(NCHUNK - 1, 1)
    wait_scatter(1)
    issue_gather(1, 1)
    compute(0)
    issue_scatter(0)
    wait_gather(1, 1)
    wait_e(1)
    _stage_dst(1, 1)
    compute(1)
    issue_scatter(1)
    for u in (2, 3, 0, 1):
        wait_scatter(u)

    plsc.subcore_barrier()
    for k in range(KMAX):
        rc = sid + k * NS
        p = k % 2
        if k >= 2:
            rc2 = sid + (k - 2) * NS

            @pl.when(rc2 < NRC)
            def _():
                pltpu.make_async_copy(
                    gv[p], out_hbm.at[cid, pl.ds(0, RZ), :], idsem[p]).wait()

        @pl.when(rc < NRC)
        def _():
            pltpu.async_copy(agg_sh.at[pl.ds(rc * RZ, RZ), :], gv[p], isem[p])
        if k >= 1:
            rc1 = sid + (k - 1) * NS
            q = (k - 1) % 2

            @pl.when(rc1 < NRC)
            def _():
                pltpu.make_async_copy(
                    agg_sh.at[pl.ds(0, RZ), :], gv[q], isem[q]).wait()
                pltpu.async_copy(
                    gv[q], out_hbm.at[cid, pl.ds(rc1 * RZ, RZ), :], idsem[q])
    rcl = sid + (KMAX - 1) * NS
    ql = (KMAX - 1) % 2

    @pl.when(rcl < NRC)
    def _():
        pltpu.make_async_copy(agg_sh.at[pl.ds(0, RZ), :], gv[ql], isem[ql]).wait()
        pltpu.async_copy(
            gv[ql], out_hbm.at[cid, pl.ds(rcl * RZ, RZ), :], idsem[ql])
    for kk in (KMAX - 2, KMAX - 1):
        rck = sid + kk * NS
        pq = kk % 2

        @pl.when(rck < NRC)
        def _():
            pltpu.make_async_copy(
                gv[pq], out_hbm.at[cid, pl.ds(0, RZ), :], idsem[pq]).wait()


_sc_agg = functools.partial(
    pl.kernel,
    out_type=jax.ShapeDtypeStruct((NC, N, D), jnp.float32),
    mesh=plsc.VectorSubcoreMesh(core_axis_name="c", subcore_axis_name="s"),
    scratch_types=(
        [pltpu.VMEM((C,), jnp.int32)] * 8
        + [pltpu.VMEM((C, D), jnp.float32)] * 8
        + [pltpu.VMEM_SHARED((N, D), jnp.float32)]
        + [pltpu.SemaphoreType.DMA] * 16
    ),
)(_sc_agg_body)


def _c1_body(p_ref, h_ref, h0_ref, b_ref, eps_ref,
             y_ref, s1_ref, sq_ref, cnt_ref):
    i = pl.program_id(0)
    agg = p_ref[0] + p_ref[1]
    x = agg + (1.0 + eps_ref[0, 0]) * h_ref[...]
    x = x * (1.0 / (1.0 + jnp.exp(-x)))
    y = x + h0_ref[...]
    y_ref[...] = y
    b = b_ref[0, 0, :]
    onehot = (b[:, None] == lax.broadcasted_iota(jnp.int32, (1, G), 1)
              ).astype(jnp.float32)
    s1 = lax.dot_general(onehot, y, (((0,), (0,)), ((), ())),
                         preferred_element_type=jnp.float32)
    sq = lax.dot_general(onehot, y * y, (((0,), (0,)), ((), ())),
                         preferred_element_type=jnp.float32)
    cnt = lax.dot_general(onehot, jnp.ones((R, D), jnp.float32),
                          (((0,), (0,)), ((), ())),
                          preferred_element_type=jnp.float32)

    @pl.when(i == 0)
    def _():
        s1_ref[...] = jnp.zeros_like(s1_ref)
        sq_ref[...] = jnp.zeros_like(sq_ref)
        cnt_ref[...] = jnp.zeros_like(cnt_ref)

    s1_ref[...] += s1
    sq_ref[...] += sq
    cnt_ref[...] += cnt


def _c2_body(y_ref, b_ref, s1_ref, sq_ref, cnt_ref, sc_ref, w_ref, bias_ref,
             o_ref):
    n = jnp.maximum(cnt_ref[...], 1.0)
    mean = s1_ref[...] / n
    ms = mean * sc_ref[...]
    var = sq_ref[...] / n - 2.0 * ms * mean + ms * ms
    winv = w_ref[...] * lax.rsqrt(var + 1e-5)
    b = b_ref[0, 0, :]
    onehot = (b[:, None] == lax.broadcasted_iota(jnp.int32, (1, G), 1)
              ).astype(jnp.float32)
    ms_r = lax.dot_general(onehot, ms, (((1,), (0,)), ((), ())),
                           preferred_element_type=jnp.float32)
    winv_r = lax.dot_general(onehot, winv, (((1,), (0,)), ((), ())),
                             preferred_element_type=jnp.float32)
    o_ref[...] = (y_ref[...] - ms_r) * winv_r + bias_ref[...]


def _node_phase(parts, h, node_h, batch, eps_gine,
                gn_weight, gn_bias, gn_mean_scale):
    batch3 = batch.reshape(NB, 1, R)
    eps2 = eps_gine.reshape(1, 1)
    y, s1, sq, cnt = pl.pallas_call(
        _c1_body,
        grid=(NB,),
        in_specs=[
            pl.BlockSpec((NC, R, D), lambda i: (0, i, 0)),
            pl.BlockSpec((R, D), lambda i: (i, 0)),
            pl.BlockSpec((R, D), lambda i: (i, 0)),
            pl.BlockSpec((1, 1, R), lambda i: (i, 0, 0)),
            pl.BlockSpec((1, 1), lambda i: (0, 0)),
        ],
        out_specs=[
            pl.BlockSpec((R, D), lambda i: (i, 0)),
            pl.BlockSpec((G, D), lambda i: (0, 0)),
            pl.BlockSpec((G, D), lambda i: (0, 0)),
            pl.BlockSpec((G, D), lambda i: (0, 0)),
        ],
        out_shape=[
            jax.ShapeDtypeStruct((N, D), jnp.float32),
            jax.ShapeDtypeStruct((G, D), jnp.float32),
            jax.ShapeDtypeStruct((G, D), jnp.float32),
            jax.ShapeDtypeStruct((G, D), jnp.float32),
        ],
    )(parts, h, node_h, batch3, eps2)

    return pl.pallas_call(
        _c2_body,
        grid=(NB,),
        in_specs=[
            pl.BlockSpec((R, D), lambda i: (i, 0)),
            pl.BlockSpec((1, 1, R), lambda i: (i, 0, 0)),
            pl.BlockSpec((G, D), lambda i: (0, 0)),
            pl.BlockSpec((G, D), lambda i: (0, 0)),
            pl.BlockSpec((G, D), lambda i: (0, 0)),
            pl.BlockSpec((1, D), lambda i: (0, 0)),
            pl.BlockSpec((1, D), lambda i: (0, 0)),
            pl.BlockSpec((1, D), lambda i: (0, 0)),
        ],
        out_specs=pl.BlockSpec((R, D), lambda i: (i, 0)),
        out_shape=jax.ShapeDtypeStruct((N, D), jnp.float32),
    )(y, batch3, s1, sq, cnt, gn_mean_scale.reshape(1, D),
      gn_weight.reshape(1, D), gn_bias.reshape(1, D))


def kernel(node_h, edge_attr, batch, edge_index, W1, b1, Wl, bl, eps_gine,
           gn_weight, gn_bias, gn_mean_scale):
    h = _node_mm(node_h, W1, b1)
    e = _edge_mm(edge_attr, Wl, bl)
    parts = _sc_agg(h, e, edge_index[0], edge_index[1])
    return _node_phase(parts, h, node_h, batch, eps_gine,
                       gn_weight, gn_bias, gn_mean_scale)
